# baseline (speedup 1.0000x reference)
import jax
import jax.numpy as jnp
from jax.experimental import pallas as pl
from jax.experimental.pallas import tpu as pltpu

_BT = 2048
_NSPLIT = 4


def _mlp_kernel(*refs):
    x_refs = refs[:_NSPLIT]
    w1_refs = refs[_NSPLIT:2 * _NSPLIT]
    b1_ref, w2_ref, b2_ref, w3_ref, b3_ref, o_ref = refs[2 * _NSPLIT:]
    h = b1_ref[...]
    for xr, wr in zip(x_refs, w1_refs):
        h = h + jnp.dot(xr[...].astype(jnp.bfloat16),
                        wr[...].astype(jnp.bfloat16),
                        preferred_element_type=jnp.float32)
    h = jnp.maximum(h, 0.0).astype(jnp.bfloat16)
    h = jnp.dot(h, w2_ref[...].astype(jnp.bfloat16),
                preferred_element_type=jnp.float32) + b2_ref[...]
    h = jnp.maximum(h, 0.0).astype(jnp.bfloat16)
    o_ref[...] = jnp.dot(h, w3_ref[...].astype(jnp.bfloat16),
                         preferred_element_type=jnp.float32) + b3_ref[...]


def kernel(hidden_states, W1, b1, W2, b2, W3, b3):
    x = hidden_states
    if x.ndim == 3:
        x = jnp.mean(x, axis=1)
    n, d = x.shape
    e = W3.shape[1]
    dk = d // _NSPLIT
    x_specs = [
        pl.BlockSpec((_BT, dk), lambda i, k=k: (i, k)) for k in range(_NSPLIT)
    ]
    w1_specs = [
        pl.BlockSpec((dk, W1.shape[1]), lambda i, k=k: (k, 0))
        for k in range(_NSPLIT)
    ]
    return pl.pallas_call(
        _mlp_kernel,
        grid=(n // _BT,),
        in_specs=x_specs + w1_specs + [
            pl.BlockSpec((1, b1.shape[0]), lambda i: (0, 0)),
            pl.BlockSpec(W2.shape, lambda i: (0, 0)),
            pl.BlockSpec((1, b2.shape[0]), lambda i: (0, 0)),
            pl.BlockSpec(W3.shape, lambda i: (0, 0)),
            pl.BlockSpec((1, b3.shape[0]), lambda i: (0, 0)),
        ],
        out_specs=pl.BlockSpec((_BT, e), lambda i: (i, 0)),
        out_shape=jax.ShapeDtypeStruct((n, e), jnp.float32),
        compiler_params=pltpu.CompilerParams(
            dimension_semantics=("arbitrary",),
        ),
    )(*([x] * _NSPLIT), *([W1] * _NSPLIT),
      b1.reshape(1, -1), W2, b2.reshape(1, -1), W3, b3.reshape(1, -1))
